# baseline (speedup 1.0000x reference)
import functools

import jax
import jax.numpy as jnp
from jax import lax
from jax.experimental import pallas as pl
from jax.experimental.pallas import tpu as pltpu
from jax.experimental.pallas import tpu_sc as plsc

_NC = 2
_NS = 16
_NW = _NC * _NS

_CHUNK = 128


def _embed_gather(idx3, table):
    n_chunks, chunk = idx3.shape[1], idx3.shape[2]
    d = table.shape[1]
    per_w = n_chunks * chunk
    mesh = plsc.VectorSubcoreMesh(core_axis_name="c", subcore_axis_name="s")

    @functools.partial(
        pl.kernel,
        mesh=mesh,
        compiler_params=pltpu.CompilerParams(use_tc_tiling_on_sc=False),
        out_type=jax.ShapeDtypeStruct((_NW, per_w, d), jnp.float32),
        scratch_types=[
            pltpu.VMEM((n_chunks, chunk), jnp.int32),
            pltpu.VMEM((chunk, d), jnp.float32),
            pltpu.VMEM((chunk, d), jnp.float32),
            pltpu.SemaphoreType.DMA,
            pltpu.SemaphoreType.DMA,
            pltpu.SemaphoreType.DMA,
            pltpu.SemaphoreType.DMA,
        ],
    )
    def k(idx_hbm, table_hbm, out_hbm, idx_v, buf0, buf1, g0, g1, o0, o1):
        wid = lax.axis_index("s") * _NC + lax.axis_index("c")
        pltpu.sync_copy(idx_hbm.at[wid], idx_v)
        bufs = (buf0, buf1)
        gsems = (g0, g1)
        osems = (o0, o1)
        gcp = [None, None]
        ocp = [None, None]
        gcp[0] = pltpu.async_copy(table_hbm.at[idx_v.at[0]], bufs[0], gsems[0])
        for i in range(n_chunks):
            cur = i & 1
            nxt = i + 1
            if nxt < n_chunks:
                nb = nxt & 1
                if ocp[nb] is not None:
                    ocp[nb].wait()
                gcp[nb] = pltpu.async_copy(
                    table_hbm.at[idx_v.at[nxt]], bufs[nb], gsems[nb])
            gcp[cur].wait()
            ocp[cur] = pltpu.async_copy(
                bufs[cur], out_hbm.at[wid, pl.ds(i * chunk, chunk)], osems[cur])
        for cur in range(2):
            if ocp[cur] is not None:
                ocp[cur].wait()

    return k(idx3, table)


def kernel(indices, table):
    b, p = indices.shape
    d = table.shape[1]
    n = b * p
    per_w = n // _NW
    n_chunks = per_w // _CHUNK
    idx3 = indices.astype(jnp.int32).reshape(_NW, n_chunks, _CHUNK)
    out = _embed_gather(idx3, table)
    return out.reshape(b, p, d)


# speedup vs baseline: 3.0828x; 3.0828x over previous
import functools

import jax
import jax.numpy as jnp
from jax import lax
from jax.experimental import pallas as pl
from jax.experimental.pallas import tpu as pltpu
from jax.experimental.pallas import tpu_sc as plsc

_NC = 2
_NS = 16
_NW = _NC * _NS

_S = 8
_P = 20


def _embed_gather(idx3, table, b):
    n_chunks = idx3.shape[1]
    sg = idx3.shape[2]
    d = table.shape[1]
    per_w_sent = n_chunks * _S
    mesh = plsc.VectorSubcoreMesh(core_axis_name="c", subcore_axis_name="s")

    @functools.partial(
        pl.kernel,
        mesh=mesh,
        compiler_params=pltpu.CompilerParams(use_tc_tiling_on_sc=True),
        out_type=jax.ShapeDtypeStruct((b, _P, d), jnp.float32),
        scratch_types=[
            pltpu.VMEM((sg,), jnp.int32),
            pltpu.VMEM((_S, _P, d), jnp.float32),
            pltpu.SemaphoreType.DMA,
        ],
    )
    def k(idx_hbm, table_hbm, out_hbm, idx_v, buf, sem):
        wid = lax.axis_index("s") * _NC + lax.axis_index("c")

        def chunk(c, carry):
            pltpu.sync_copy(idx_hbm.at[wid, c], idx_v)
            cps = []
            for g in range(sg // 16):
                vec = idx_v[pl.ds(g * 16, 16)]
                for j in range(16):
                    r = g * 16 + j
                    row = vec[j]
                    s, t = r // _P, r % _P
                    cps.append(pltpu.async_copy(
                        table_hbm.at[row], buf.at[s, t], sem))
            for cp in cps:
                cp.wait()
            sent0 = wid * per_w_sent + c * _S
            pltpu.sync_copy(buf, out_hbm.at[pl.ds(sent0, _S)])
            return carry

        lax.fori_loop(0, n_chunks, chunk, 0)

    return k(idx3, table)


def kernel(indices, table):
    b, p = indices.shape
    n = b * p
    per_w = n // _NW
    n_chunks = per_w // (_S * _P)
    idx3 = indices.astype(jnp.int32).reshape(_NW, n_chunks, _S * _P)
    return _embed_gather(idx3, table, b)


# speedup vs baseline: 3.3827x; 1.0973x over previous
import functools

import jax
import jax.numpy as jnp
from jax import lax
from jax.experimental import pallas as pl
from jax.experimental.pallas import tpu as pltpu
from jax.experimental.pallas import tpu_sc as plsc

_NC = 2
_NS = 16
_NW = _NC * _NS

_S = 8
_P = 20
_NBUF = 4


def _embed_gather(idx2, table, b):
    per_w = idx2.shape[1]
    d = table.shape[1]
    sg = _S * _P
    n_chunks = per_w // sg
    mesh = plsc.VectorSubcoreMesh(core_axis_name="c", subcore_axis_name="s")

    @functools.partial(
        pl.kernel,
        mesh=mesh,
        compiler_params=pltpu.CompilerParams(use_tc_tiling_on_sc=True),
        out_type=jax.ShapeDtypeStruct((b, _P, d), jnp.float32),
        scratch_types=[
            pltpu.VMEM((per_w,), jnp.int32),
            *[pltpu.VMEM((_S, _P, d), jnp.float32) for _ in range(_NBUF)],
            *[pltpu.SemaphoreType.DMA for _ in range(2 * _NBUF)],
        ],
    )
    def k(idx_hbm, table_hbm, out_hbm, idx_v, *bufs_sems):
        bufs = bufs_sems[:_NBUF]
        gsems = bufs_sems[_NBUF:2 * _NBUF]
        osems = bufs_sems[2 * _NBUF:]
        wid = lax.axis_index("s") * _NC + lax.axis_index("c")
        sent_base = wid * (per_w // _P)
        pltpu.sync_copy(idx_hbm.at[wid], idx_v)

        def issue(c, bi):
            for g in range(sg // 16):
                vec = idx_v[pl.ds(c * sg + g * 16, 16)]
                for j in range(16):
                    r = g * 16 + j
                    pltpu.async_copy(
                        table_hbm.at[vec[j]], bufs[bi].at[r // _P, r % _P],
                        gsems[bi])

        def drain(bi):
            for _ in range(sg):
                pltpu.make_async_copy(
                    table_hbm.at[0], bufs[bi].at[0, 0], gsems[bi]).wait()

        for bi in range(_NBUF):
            issue(bi, bi)

        def body(cp, carry):
            c0 = cp * _NBUF
            for bi in range(_NBUF):
                drain(bi)
                pltpu.async_copy(
                    bufs[bi],
                    out_hbm.at[pl.ds(sent_base + (c0 + bi) * _S, _S)],
                    osems[bi])
            for bi in range(_NBUF):
                c_next = c0 + _NBUF + bi

                @pl.when(c_next < n_chunks)
                def _():
                    pltpu.make_async_copy(
                        bufs[bi], out_hbm.at[pl.ds(0, _S)], osems[bi]).wait()
                    issue(c_next, bi)

            return carry

        lax.fori_loop(0, n_chunks // _NBUF, body, 0)
        for bi in range(_NBUF):
            pltpu.make_async_copy(
                bufs[bi], out_hbm.at[pl.ds(0, _S)], osems[bi]).wait()

    return k(idx2, table)


def kernel(indices, table):
    b, p = indices.shape
    n = b * p
    per_w = n // _NW
    idx2 = indices.astype(jnp.int32).reshape(_NW, per_w)
    return _embed_gather(idx2, table, b)
